# Initial kernel scaffold; baseline (speedup 1.0000x reference)
import jax
import jax.numpy as jnp
from jax.experimental import pallas as pl


def kernel(node_states, edges, kernel, kernel_attention):
    raise NotImplementedError("write your pallas kernel here")


# speedup vs baseline: 19.7859x; 19.7859x over previous
import jax
import jax.numpy as jnp
from jax import lax
from jax.experimental import pallas as pl
from jax.experimental.pallas import tpu as pltpu
from jax.experimental.pallas import tpu_sc as plsc

N = 10000
E = 320000
D = 128
W144 = 144
CH = 125
NW = 32
EPW = E // NW
NCH = EPW // CH
SGRP = 8
STRIPE = 632
NPAD = 16 * STRIPE

_SC_PARAMS = pltpu.CompilerParams(
    needs_layout_passes=False, use_tc_tiling_on_sc=False)


def _mm_body(x_ref, w_ref, ka2_ref, h_ref, ab_ref):
    hb = jnp.dot(x_ref[...], w_ref[...], preferred_element_type=jnp.float32)
    h_ref[:, 0:D] = hb
    col = lax.broadcasted_iota(jnp.int32, (hb.shape[0], 16), 1)
    h_ref[:, D:W144] = jnp.where(col == 0, 1.0, 0.0)
    ab_ref[...] = jnp.dot(hb, ka2_ref[...], preferred_element_type=jnp.float32)


def _score_body(srcf_hbm, dstf_hbm, asrc_hbm, adst_hbm, s_hbm,
                srcf_v, dstf_v, asrc_v, adst_v, s_v):
    c = lax.axis_index("c")
    s_id = lax.axis_index("s")
    base = (s_id * 2 + c) * EPW

    pltpu.sync_copy(srcf_hbm.at[pl.ds(base, EPW)], srcf_v)
    pltpu.sync_copy(dstf_hbm.at[pl.ds(base, EPW)], dstf_v)
    pltpu.sync_copy(asrc_hbm, asrc_v)
    pltpu.sync_copy(adst_hbm, adst_v)

    def _score(i, _):
        si = srcf_v[pl.ds(i * 16, 16)]
        di = dstf_v[pl.ds(i * 16, 16)]
        raw = plsc.load_gather(asrc_v, [si]) + plsc.load_gather(adst_v, [di])
        lk = jnp.maximum(raw, raw * 0.2)
        s_v[pl.ds(i * 16, 16)] = jnp.exp(jnp.clip(lk, -2.0, 2.0))
        return 0
    lax.fori_loop(0, EPW // 16, _score, 0)

    pltpu.sync_copy(s_v, s_hbm.at[pl.ds(base, EPW)])


def _agg_body(h_hbm, src2_hbm, dst2_hbm, s_hbm, acc0_hbm, acc1_hbm,
              src2_v, dst2_v, s8_v, rows_v, acc_sh, sem):
    c = lax.axis_index("c")
    s_id = lax.axis_index("s")
    w = s_id * 2 + c
    base = w * EPW

    pltpu.sync_copy(src2_hbm.at[pl.ds(w * NCH, NCH)], src2_v)
    pltpu.sync_copy(dst2_hbm.at[pl.ds(w * NCH, NCH)], dst2_v)

    def _zrow(k, _):
        for i in range(W144 // 16):
            rows_v[k, pl.ds(i * 16, 16)] = jnp.zeros((16,), jnp.float32)
        return 0
    lax.fori_loop(0, CH, _zrow, 0)
    row0 = s_id * STRIPE
    for off, n in ((0, 120), (120, 120), (240, 120), (360, 120), (480, 120),
                   (600, 32)):
        pltpu.sync_copy(rows_v.at[pl.ds(0, n)],
                        acc_sh.at[pl.ds(row0 + off, n)])

    plsc.subcore_barrier()

    def _group(g, _):
        pltpu.sync_copy(s_hbm.at[pl.ds(base + g * (SGRP * CH), SGRP * CH)],
                        s8_v)
        for jj in range(SGRP):
            j = g * SGRP + jj
            pltpu.async_copy(h_hbm.at[dst2_v.at[j]], rows_v, sem).wait()

            def _scale(k, _):
                sc = plsc.load_gather(
                    s8_v, [jnp.full((16,), jj * CH + k, jnp.int32)])
                for i in range(W144 // 16):
                    sl = pl.ds(i * 16, 16)
                    rows_v[k, sl] = rows_v[k, sl] * sc
                return 0
            lax.fori_loop(0, CH, _scale, 0)
            pltpu.sync_copy(rows_v, acc_sh.at[src2_v.at[j]], add=True)
        return 0
    lax.fori_loop(0, NCH // SGRP, _group, 0)

    plsc.subcore_barrier()

    @pl.when(c == 0)
    def _():
        pltpu.sync_copy(acc_sh.at[pl.ds(row0, STRIPE)],
                        acc0_hbm.at[pl.ds(row0, STRIPE)])

    @pl.when(c == 1)
    def _():
        pltpu.sync_copy(acc_sh.at[pl.ds(row0, STRIPE)],
                        acc1_hbm.at[pl.ds(row0, STRIPE)])


def _combine_body(a0_ref, a1_ref, out_ref):
    t = a0_ref[...] + a1_ref[...]
    num = t[:, 0:D]
    den = t[:, D:D + 1]
    safe = jnp.where(den > 0.0, den, 1.0)
    out_ref[...] = num / safe


def kernel(node_states, edges, kernel, kernel_attention):
    ka2 = jnp.concatenate(
        [kernel_attention[:D], kernel_attention[D:]], axis=1)

    blk = 2000
    h144, ab = pl.pallas_call(
        _mm_body,
        grid=(N // blk,),
        in_specs=[
            pl.BlockSpec((blk, D), lambda i: (i, 0)),
            pl.BlockSpec((D, D), lambda i: (0, 0)),
            pl.BlockSpec((D, 2), lambda i: (0, 0)),
        ],
        out_specs=[
            pl.BlockSpec((blk, W144), lambda i: (i, 0)),
            pl.BlockSpec((blk, 2), lambda i: (i, 0)),
        ],
        out_shape=[
            jax.ShapeDtypeStruct((N, W144), jnp.float32),
            jax.ShapeDtypeStruct((N, 2), jnp.float32),
        ],
    )(node_states, kernel, ka2)

    src = edges[:, 0]
    dst = edges[:, 1]
    src2 = src.reshape(E // CH, CH)
    dst2 = dst.reshape(E // CH, CH)

    mesh = plsc.VectorSubcoreMesh(core_axis_name="c", subcore_axis_name="s")

    s_all = pl.kernel(
        _score_body,
        out_type=jax.ShapeDtypeStruct((E,), jnp.float32),
        mesh=mesh,
        compiler_params=_SC_PARAMS,
        scratch_types=[
            pltpu.VMEM((EPW,), jnp.int32),
            pltpu.VMEM((EPW,), jnp.int32),
            pltpu.VMEM((N,), jnp.float32),
            pltpu.VMEM((N,), jnp.float32),
            pltpu.VMEM((EPW,), jnp.float32),
        ],
    )(src, dst, ab[:, 0], ab[:, 1])

    acc0, acc1 = pl.kernel(
        _agg_body,
        out_type=[
            jax.ShapeDtypeStruct((NPAD, W144), jnp.float32),
            jax.ShapeDtypeStruct((NPAD, W144), jnp.float32),
        ],
        mesh=mesh,
        compiler_params=_SC_PARAMS,
        scratch_types=[
            pltpu.VMEM((NCH, CH), jnp.int32),
            pltpu.VMEM((NCH, CH), jnp.int32),
            pltpu.VMEM((SGRP * CH,), jnp.float32),
            pltpu.VMEM((CH, W144), jnp.float32),
            pltpu.VMEM_SHARED((NPAD, W144), jnp.float32),
            pltpu.SemaphoreType.DMA,
        ],
    )(h144, src2, dst2, s_all)

    out = pl.pallas_call(
        _combine_body,
        grid=(N // blk,),
        in_specs=[
            pl.BlockSpec((blk, W144), lambda i: (i, 0)),
            pl.BlockSpec((blk, W144), lambda i: (i, 0)),
        ],
        out_specs=pl.BlockSpec((blk, D), lambda i: (i, 0)),
        out_shape=jax.ShapeDtypeStruct((N, D), jnp.float32),
    )(acc0, acc1)
    return out


# speedup vs baseline: 24.1771x; 1.2219x over previous
import jax
import jax.numpy as jnp
from jax import lax
from jax.experimental import pallas as pl
from jax.experimental.pallas import tpu as pltpu
from jax.experimental.pallas import tpu_sc as plsc

N = 10000
E = 320000
D = 128
W144 = 144
CH = 50
NW = 32
EPW = E // NW
NCH = EPW // CH
SGRP = 8
NG = NCH // SGRP
STRIPE = 632
NPAD = 16 * STRIPE

_SC_PARAMS = pltpu.CompilerParams(
    needs_layout_passes=False, use_tc_tiling_on_sc=False)


def _mm_body(x_ref, w_ref, ka2_ref, h_ref, ab_ref):
    hb = jnp.dot(x_ref[...], w_ref[...], preferred_element_type=jnp.float32)
    h_ref[:, 0:D] = hb
    col = lax.broadcasted_iota(jnp.int32, (hb.shape[0], 16), 1)
    h_ref[:, D:W144] = jnp.where(col == 0, 1.0, 0.0)
    ab_ref[...] = jnp.dot(hb, ka2_ref[...], preferred_element_type=jnp.float32)


def _score_body(srcf_hbm, dstf_hbm, asrc_hbm, adst_hbm, s_hbm,
                srcf_v, dstf_v, asrc_v, adst_v, s_v):
    c = lax.axis_index("c")
    s_id = lax.axis_index("s")
    base = (s_id * 2 + c) * EPW

    pltpu.sync_copy(srcf_hbm.at[pl.ds(base, EPW)], srcf_v)
    pltpu.sync_copy(dstf_hbm.at[pl.ds(base, EPW)], dstf_v)
    pltpu.sync_copy(asrc_hbm, asrc_v)
    pltpu.sync_copy(adst_hbm, adst_v)

    def _score(i, _):
        si = srcf_v[pl.ds(i * 16, 16)]
        di = dstf_v[pl.ds(i * 16, 16)]
        raw = plsc.load_gather(asrc_v, [si]) + plsc.load_gather(adst_v, [di])
        lk = jnp.maximum(raw, raw * 0.2)
        s_v[pl.ds(i * 16, 16)] = jnp.exp(jnp.clip(lk, -2.0, 2.0))
        return 0
    lax.fori_loop(0, EPW // 16, _score, 0)

    pltpu.sync_copy(s_v, s_hbm.at[pl.ds(base, EPW)])


def _agg_body(h_hbm, src2_hbm, dst2_hbm, s_hbm, acc0_hbm, acc1_hbm,
              src2_v, dst2_v, sg_v, rows0_v, rows1_v, acc_sh,
              sem_g0, sem_g1, sem_s0, sem_s1, sem_sg):
    c = lax.axis_index("c")
    s_id = lax.axis_index("s")
    w = s_id * 2 + c
    base = w * EPW
    rows = (rows0_v, rows1_v)
    sem_g = (sem_g0, sem_g1)
    sem_s = (sem_s0, sem_s1)
    SB = SGRP * CH

    pltpu.sync_copy(src2_hbm.at[pl.ds(w * NCH, NCH)], src2_v)
    pltpu.sync_copy(dst2_hbm.at[pl.ds(w * NCH, NCH)], dst2_v)

    def _zrow(k, _):
        for i in range(W144 // 16):
            rows0_v[k, pl.ds(i * 16, 16)] = jnp.zeros((16,), jnp.float32)
        return 0
    lax.fori_loop(0, CH, _zrow, 0)
    row0 = s_id * STRIPE
    for off in range(0, STRIPE - 8, 48):
        pltpu.sync_copy(rows0_v.at[pl.ds(0, 48)],
                        acc_sh.at[pl.ds(row0 + off, 48)])
    pltpu.sync_copy(rows0_v.at[pl.ds(0, 8)],
                    acc_sh.at[pl.ds(row0 + STRIPE - 8, 8)])

    plsc.subcore_barrier()

    def _wait_gather(p):
        pltpu.make_async_copy(h_hbm.at[pl.ds(0, CH)], rows[p],
                              sem_g[p]).wait()

    def _wait_scatter(p):
        pltpu.make_async_copy(rows[p], acc_sh.at[pl.ds(0, CH)],
                              sem_s[p]).wait()

    pltpu.sync_copy(s_hbm.at[pl.ds(base, SB)], sg_v.at[pl.ds(0, SB)])
    pltpu.async_copy(h_hbm.at[dst2_v.at[0]], rows0_v, sem_g0)

    def _group(g, _):
        nxt_off = pl.multiple_of(((g + 1) % 2) * SB, 8)

        @pl.when(g < NG - 1)
        def _():
            pltpu.async_copy(
                s_hbm.at[pl.ds(base + (g + 1) * SB, SB)],
                sg_v.at[pl.ds(nxt_off, SB)], sem_sg)

        s_off = (g % 2) * SB
        for jj in range(SGRP):
            j = g * SGRP + jj
            p = jj % 2
            q = 1 - p
            _wait_gather(p)
            if jj == 0:
                @pl.when(g > 0)
                def _():
                    _wait_scatter(q)
                pltpu.async_copy(h_hbm.at[dst2_v.at[j + 1]], rows[q],
                                 sem_g[q])
            elif jj < SGRP - 1:
                _wait_scatter(q)
                pltpu.async_copy(h_hbm.at[dst2_v.at[j + 1]], rows[q],
                                 sem_g[q])
            else:
                @pl.when(g < NG - 1)
                def _():
                    _wait_scatter(q)
                    pltpu.async_copy(h_hbm.at[dst2_v.at[j + 1]], rows[q],
                                     sem_g[q])

            def _scale(k, _):
                sc = plsc.load_gather(
                    sg_v, [jnp.full((16,), s_off + jj * CH, jnp.int32) + k])
                for i in range(W144 // 16):
                    sl = pl.ds(i * 16, 16)
                    rows[p][k, sl] = rows[p][k, sl] * sc
                return 0
            lax.fori_loop(0, CH, _scale, 0)
            pltpu.async_copy(rows[p], acc_sh.at[src2_v.at[j]], sem_s[p],
                             add=True)

        @pl.when(g < NG - 1)
        def _():
            pltpu.make_async_copy(s_hbm.at[pl.ds(0, SB)],
                                  sg_v.at[pl.ds(0, SB)], sem_sg).wait()
        return 0

    lax.fori_loop(0, NG, _group, 0)

    _wait_scatter(0)
    _wait_scatter(1)

    plsc.subcore_barrier()

    @pl.when(c == 0)
    def _():
        pltpu.sync_copy(acc_sh.at[pl.ds(row0, STRIPE)],
                        acc0_hbm.at[pl.ds(row0, STRIPE)])

    @pl.when(c == 1)
    def _():
        pltpu.sync_copy(acc_sh.at[pl.ds(row0, STRIPE)],
                        acc1_hbm.at[pl.ds(row0, STRIPE)])


def _combine_body(a0_ref, a1_ref, out_ref):
    t = a0_ref[...] + a1_ref[...]
    num = t[:, 0:D]
    den = t[:, D:D + 1]
    safe = jnp.where(den > 0.0, den, 1.0)
    out_ref[...] = num / safe


def kernel(node_states, edges, kernel, kernel_attention):
    ka2 = jnp.concatenate(
        [kernel_attention[:D], kernel_attention[D:]], axis=1)

    blk = 2000
    h144, ab = pl.pallas_call(
        _mm_body,
        grid=(N // blk,),
        in_specs=[
            pl.BlockSpec((blk, D), lambda i: (i, 0)),
            pl.BlockSpec((D, D), lambda i: (0, 0)),
            pl.BlockSpec((D, 2), lambda i: (0, 0)),
        ],
        out_specs=[
            pl.BlockSpec((blk, W144), lambda i: (i, 0)),
            pl.BlockSpec((blk, 2), lambda i: (i, 0)),
        ],
        out_shape=[
            jax.ShapeDtypeStruct((N, W144), jnp.float32),
            jax.ShapeDtypeStruct((N, 2), jnp.float32),
        ],
    )(node_states, kernel, ka2)

    src = edges[:, 0]
    dst = edges[:, 1]
    src2 = src.reshape(E // CH, CH)
    dst2 = dst.reshape(E // CH, CH)

    mesh = plsc.VectorSubcoreMesh(core_axis_name="c", subcore_axis_name="s")

    s_all = pl.kernel(
        _score_body,
        out_type=jax.ShapeDtypeStruct((E,), jnp.float32),
        mesh=mesh,
        compiler_params=_SC_PARAMS,
        scratch_types=[
            pltpu.VMEM((EPW,), jnp.int32),
            pltpu.VMEM((EPW,), jnp.int32),
            pltpu.VMEM((N,), jnp.float32),
            pltpu.VMEM((N,), jnp.float32),
            pltpu.VMEM((EPW,), jnp.float32),
        ],
    )(src, dst, ab[:, 0], ab[:, 1])

    acc0, acc1 = pl.kernel(
        _agg_body,
        out_type=[
            jax.ShapeDtypeStruct((NPAD, W144), jnp.float32),
            jax.ShapeDtypeStruct((NPAD, W144), jnp.float32),
        ],
        mesh=mesh,
        compiler_params=_SC_PARAMS,
        scratch_types=[
            pltpu.VMEM((NCH, CH), jnp.int32),
            pltpu.VMEM((NCH, CH), jnp.int32),
            pltpu.VMEM((2 * SGRP * CH,), jnp.float32),
            pltpu.VMEM((CH, W144), jnp.float32),
            pltpu.VMEM((CH, W144), jnp.float32),
            pltpu.VMEM_SHARED((NPAD, W144), jnp.float32),
            pltpu.SemaphoreType.DMA,
            pltpu.SemaphoreType.DMA,
            pltpu.SemaphoreType.DMA,
            pltpu.SemaphoreType.DMA,
            pltpu.SemaphoreType.DMA,
        ],
    )(h144, src2, dst2, s_all)

    out = pl.pallas_call(
        _combine_body,
        grid=(N // blk,),
        in_specs=[
            pl.BlockSpec((blk, W144), lambda i: (i, 0)),
            pl.BlockSpec((blk, W144), lambda i: (i, 0)),
        ],
        out_specs=pl.BlockSpec((blk, D), lambda i: (i, 0)),
        out_shape=jax.ShapeDtypeStruct((N, D), jnp.float32),
    )(acc0, acc1)
    return out
